# baseline (speedup 1.0000x reference)
import functools

import jax
import jax.numpy as jnp
import numpy as np
from jax import lax
from jax.experimental import pallas as pl
from jax.experimental.pallas import tpu as pltpu
from jax.experimental.pallas import tpu_sc as plsc

NC = 2
NS = 16
NW = NC * NS
LANES = 16
CHUNK = 128
PKBITS = 14


def _pre_body(x_ref, w_ref, al_ref, ar_ref, z_ref, sl_ref, sr_ref, cv_ref):
    x = x_ref[...]
    z = lax.dot_general(x, w_ref[...], (((1,), (1,)), ((), ())),
                        preferred_element_type=jnp.float32)
    z_ref[...] = z
    sl = jnp.sum(z * al_ref[...][None, :], axis=1)
    sr = jnp.sum(z * ar_ref[...][None, :], axis=1)
    sl_ref[...] = sl
    sr_ref[...] = sr
    c_sh = jnp.maximum(jnp.max(sl) + jnp.max(sr), 0.0)
    cv_ref[...] = jnp.full((LANES,), c_sh, jnp.float32)


def _post_body(n, hp_ref, dp_ref, o_ref):
    den = dp_ref[0, :n] + dp_ref[1, :n]
    den = jnp.where(den == 0.0, 1.0, den)
    h = hp_ref[0, :n, :] + hp_ref[1, :n, :]
    o_ref[...] = h / den[:, None]


def _make_sc_kernel(n, d, n_pad, ch0, ch1):
    rows_per_tile = n_pad // NS
    zcopies = rows_per_tile // CHUNK
    chmax = max(ch0, ch1)

    mesh = plsc.VectorSubcoreMesh(core_axis_name="c", subcore_axis_name="s")

    @functools.partial(
        pl.kernel,
        out_type=[
            jax.ShapeDtypeStruct((NC, n_pad, d), jnp.float32),
            jax.ShapeDtypeStruct((NC, n_pad), jnp.float32),
        ],
        mesh=mesh,
        scratch_types=[
            pltpu.VMEM((chmax, CHUNK), jnp.int32),
            pltpu.VMEM((chmax, CHUNK), jnp.float32),
            pltpu.VMEM((CHUNK,), jnp.int32),
            pltpu.VMEM((CHUNK,), jnp.int32),
            pltpu.VMEM((CHUNK,), jnp.float32),
            pltpu.VMEM((CHUNK,), jnp.float32),
            pltpu.VMEM((CHUNK,), jnp.float32),
            pltpu.VMEM((CHUNK, d), jnp.float32),
            pltpu.VMEM((LANES,), jnp.float32),
            pltpu.VMEM_SHARED((n_pad, d), jnp.float32),
            pltpu.VMEM_SHARED((n_pad,), jnp.float32),
            pltpu.SemaphoreType.DMA,
            pltpu.SemaphoreType.DMA,
        ],
    )
    def sc_kernel(z_hbm, sl_hbm, sr_hbm, pk_hbm, w_hbm, cv_hbm,
                  h_out, den_out,
                  pk_v, w_v, src_c, dst_c, slg, srg, ex_c, rows, cv_v,
                  h_sh, den_sh, sem_s, sem_z):
        c = lax.axis_index("c")
        s = lax.axis_index("s")
        w_id = c * NS + s
        base = s * rows_per_tile

        pltpu.sync_copy(cv_hbm, cv_v)
        c_sh = cv_v[...]
        pltpu.sync_copy(pk_hbm.at[w_id], pk_v)
        pltpu.sync_copy(w_hbm.at[w_id], w_v)

        def zrow(r, _):
            for f in range(d // LANES):
                rows[r, pl.ds(f * LANES, LANES)] = jnp.zeros(
                    (LANES,), jnp.float32)
            return 0
        lax.fori_loop(0, CHUNK, zrow, 0)
        for q in range(zcopies):
            pltpu.sync_copy(rows, h_sh.at[pl.ds(base + q * CHUNK, CHUNK)])
            pltpu.sync_copy(rows.at[0],
                            den_sh.at[pl.ds(base + q * CHUNK, CHUNK)])

        plsc.subcore_barrier()

        def cbody(j, _):
            for k in range(CHUNK // LANES):
                v = pk_v[j, pl.ds(k * LANES, LANES)]
                src_c[pl.ds(k * LANES, LANES)] = lax.shift_right_logical(
                    v, PKBITS)
                dst_c[pl.ds(k * LANES, LANES)] = lax.bitwise_and(
                    v, (1 << PKBITS) - 1)
            zcp = pltpu.async_copy(z_hbm.at[src_c], rows, sem_z)
            ga = pltpu.async_copy(sl_hbm.at[src_c], slg, sem_s)
            gb = pltpu.async_copy(sr_hbm.at[dst_c], srg, sem_s)
            ga.wait()
            gb.wait()
            for k in range(CHUNK // LANES):
                wk = w_v[j, pl.ds(k * LANES, LANES)]
                raw = (slg[pl.ds(k * LANES, LANES)]
                       + srg[pl.ds(k * LANES, LANES)])
                e = wk * jnp.maximum(raw, 0.01 * raw)
                ex = jnp.where(wk >= 0.0, jnp.exp(e - c_sh), 0.0)
                ex_c[pl.ds(k * LANES, LANES)] = ex
            pltpu.sync_copy(ex_c, den_sh.at[dst_c], add=True)
            zcp.wait()
            def rblk(k, _):
                exk = ex_c[pl.ds(k * LANES, LANES)]
                for r in range(LANES):
                    a = exk[r]
                    row = k * LANES + r
                    for f in range(d // LANES):
                        v = rows[row, pl.ds(f * LANES, LANES)]
                        rows[row, pl.ds(f * LANES, LANES)] = v * a
                return 0
            lax.fori_loop(0, CHUNK // LANES, rblk, 0)
            pltpu.sync_copy(rows, h_sh.at[dst_c], add=True)
            return 0
        nch = jnp.where(c == 0, ch0, ch1)
        lax.fori_loop(0, nch, cbody, 0)

        plsc.subcore_barrier()

        pltpu.sync_copy(h_sh.at[pl.ds(base, rows_per_tile)],
                        h_out.at[c, pl.ds(base, rows_per_tile)])
        pltpu.sync_copy(den_sh.at[pl.ds(base, rows_per_tile)],
                        den_out.at[c, pl.ds(base, rows_per_tile)])

    return sc_kernel


def kernel(x, edge_index, edge_weight, W_fc, W_attn):
    n, d_in = x.shape
    d = W_fc.shape[0]
    e_cnt = edge_index.shape[1]
    assert n % LANES == 0 and d % LANES == 0

    a_l = W_attn[0, :d]
    a_r = W_attn[0, d:]

    z, sl, sr, cvec = pl.pallas_call(
        _pre_body,
        out_shape=[
            jax.ShapeDtypeStruct((n, d), jnp.float32),
            jax.ShapeDtypeStruct((n,), jnp.float32),
            jax.ShapeDtypeStruct((n,), jnp.float32),
            jax.ShapeDtypeStruct((LANES,), jnp.float32),
        ],
    )(x, W_fc, a_l, a_r)

    ch_tot = -(-e_cnt // (NS * CHUNK))
    ch0 = max(1, round(ch_tot * 0.385))
    ch1 = ch_tot - ch0
    chmax = max(ch0, ch1)
    counts = np.array([ch0] * NS + [ch1] * NS) * CHUNK
    offs = np.concatenate([[0], np.cumsum(counts)])[:NW]
    pos = offs[:, None] + np.arange(chmax * CHUNK)[None, :]
    valid = ((np.arange(chmax * CHUNK)[None, :] < counts[:, None])
             & (pos < e_cnt))
    pos_c = jnp.asarray(np.minimum(pos, e_cnt - 1))
    valid = jnp.asarray(valid)
    pk_full = (edge_index[0] << PKBITS) | edge_index[1]
    pk = jnp.where(valid, jnp.take(pk_full, pos_c),
                   0).reshape(NW, chmax, CHUNK)
    wgt = jnp.where(valid, jnp.take(edge_weight, pos_c),
                    -1.0).reshape(NW, chmax, CHUNK)

    n_pad = -(-n // (NS * CHUNK)) * NS * CHUNK
    assert n_pad < (1 << PKBITS)
    hp, dp = _make_sc_kernel(n, d, n_pad, ch0, ch1)(z, sl, sr, pk, wgt, cvec)

    out = pl.pallas_call(
        functools.partial(_post_body, n),
        out_shape=jax.ShapeDtypeStruct((n, d), jnp.float32),
    )(hp, dp)
    return out


# speedup vs baseline: 1.0742x; 1.0742x over previous
import functools

import jax
import jax.numpy as jnp
import numpy as np
from jax import lax
from jax.experimental import pallas as pl
from jax.experimental.pallas import tpu as pltpu
from jax.experimental.pallas import tpu_sc as plsc

NC = 2
NS = 16
NW = NC * NS
LANES = 16
CHUNK = 128
PKBITS = 14


def _pre_body(x_ref, w_ref, al_ref, ar_ref, z_ref, sl_ref, sr_ref, cv_ref):
    x = x_ref[...]
    z = lax.dot_general(x, w_ref[...], (((1,), (1,)), ((), ())),
                        preferred_element_type=jnp.float32)
    z_ref[...] = z
    sl = jnp.sum(z * al_ref[...][None, :], axis=1)
    sr = jnp.sum(z * ar_ref[...][None, :], axis=1)
    sl_ref[...] = sl
    sr_ref[...] = sr
    c_sh = jnp.maximum(jnp.max(sl) + jnp.max(sr), 0.0)
    cv_ref[...] = jnp.full((LANES,), c_sh, jnp.float32)


def _post_body(n, hp_ref, dp_ref, o_ref):
    den = dp_ref[0, :n] + dp_ref[1, :n]
    den = jnp.where(den == 0.0, 1.0, den)
    h = hp_ref[0, :n, :] + hp_ref[1, :n, :]
    o_ref[...] = h / den[:, None]


def _make_sc_kernel(n, d, n_pad, ch0, ch1):
    rows_per_tile = n_pad // NS
    zcopies = rows_per_tile // CHUNK
    chmax = max(ch0, ch1)

    mesh = plsc.VectorSubcoreMesh(core_axis_name="c", subcore_axis_name="s")

    @functools.partial(
        pl.kernel,
        out_type=[
            jax.ShapeDtypeStruct((NC, n_pad, d), jnp.float32),
            jax.ShapeDtypeStruct((NC, n_pad), jnp.float32),
        ],
        mesh=mesh,
        scratch_types=[
            pltpu.VMEM((chmax, CHUNK), jnp.int32),
            pltpu.VMEM((chmax, CHUNK), jnp.float32),
            pltpu.VMEM((CHUNK,), jnp.int32),
            pltpu.VMEM((CHUNK,), jnp.int32),
            pltpu.VMEM((CHUNK,), jnp.float32),
            pltpu.VMEM((CHUNK,), jnp.float32),
            pltpu.VMEM((CHUNK,), jnp.float32),
            pltpu.VMEM((CHUNK, d), jnp.float32),
            pltpu.VMEM((LANES,), jnp.float32),
            pltpu.VMEM_SHARED((n_pad, d), jnp.float32),
            pltpu.VMEM_SHARED((n_pad,), jnp.float32),
            pltpu.SemaphoreType.DMA,
            pltpu.SemaphoreType.DMA,
        ],
    )
    def sc_kernel(z_hbm, sl_hbm, sr_hbm, pk_hbm, w_hbm, cv_hbm,
                  h_out, den_out,
                  pk_v, w_v, src_c, dst_c, slg, srg, ex_c, rows, cv_v,
                  h_sh, den_sh, sem_s, sem_z):
        c = lax.axis_index("c")
        s = lax.axis_index("s")
        w_id = c * NS + s
        base = s * rows_per_tile

        pltpu.sync_copy(cv_hbm, cv_v)
        c_sh = cv_v[...]
        pltpu.sync_copy(pk_hbm.at[w_id], pk_v)
        pltpu.sync_copy(w_hbm.at[w_id], w_v)

        def zrow(r, _):
            for f in range(d // LANES):
                rows[r, pl.ds(f * LANES, LANES)] = jnp.zeros(
                    (LANES,), jnp.float32)
            return 0
        lax.fori_loop(0, CHUNK, zrow, 0)
        for q in range(zcopies):
            pltpu.sync_copy(rows, h_sh.at[pl.ds(base + q * CHUNK, CHUNK)])
            pltpu.sync_copy(rows.at[0],
                            den_sh.at[pl.ds(base + q * CHUNK, CHUNK)])

        plsc.subcore_barrier()

        def cbody(j, _):
            for k in range(CHUNK // LANES):
                v = pk_v[j, pl.ds(k * LANES, LANES)]
                src_c[pl.ds(k * LANES, LANES)] = lax.shift_right_logical(
                    v, PKBITS)
                dst_c[pl.ds(k * LANES, LANES)] = lax.bitwise_and(
                    v, (1 << PKBITS) - 1)
            zcp = pltpu.async_copy(z_hbm.at[src_c], rows, sem_z)
            ga = pltpu.async_copy(sl_hbm.at[src_c], slg, sem_s)
            gb = pltpu.async_copy(sr_hbm.at[dst_c], srg, sem_s)
            ga.wait()
            gb.wait()
            for k in range(CHUNK // LANES):
                wk = w_v[j, pl.ds(k * LANES, LANES)]
                raw = (slg[pl.ds(k * LANES, LANES)]
                       + srg[pl.ds(k * LANES, LANES)])
                e = wk * jnp.maximum(raw, 0.01 * raw)
                ex = jnp.where(wk >= 0.0, jnp.exp(e - c_sh), 0.0)
                ex_c[pl.ds(k * LANES, LANES)] = ex
            pltpu.sync_copy(ex_c, den_sh.at[dst_c], add=True)
            zcp.wait()
            def rblk(k, _):
                exk = ex_c[pl.ds(k * LANES, LANES)]
                for r in range(LANES):
                    a = exk[r]
                    row = k * LANES + r
                    for f in range(d // LANES):
                        v = rows[row, pl.ds(f * LANES, LANES)]
                        rows[row, pl.ds(f * LANES, LANES)] = v * a
                return 0
            lax.fori_loop(0, CHUNK // LANES, rblk, 0)
            pltpu.sync_copy(rows, h_sh.at[dst_c], add=True)
            return 0
        nch = jnp.where(c == 0, ch0, ch1)
        lax.fori_loop(0, nch, cbody, 0)

        plsc.subcore_barrier()

        pltpu.sync_copy(h_sh.at[pl.ds(base, rows_per_tile)],
                        h_out.at[c, pl.ds(base, rows_per_tile)])
        pltpu.sync_copy(den_sh.at[pl.ds(base, rows_per_tile)],
                        den_out.at[c, pl.ds(base, rows_per_tile)])

    return sc_kernel


def kernel(x, edge_index, edge_weight, W_fc, W_attn):
    n, d_in = x.shape
    d = W_fc.shape[0]
    e_cnt = edge_index.shape[1]
    assert n % LANES == 0 and d % LANES == 0

    a_l = W_attn[0, :d]
    a_r = W_attn[0, d:]

    z, sl, sr, cvec = pl.pallas_call(
        _pre_body,
        out_shape=[
            jax.ShapeDtypeStruct((n, d), jnp.float32),
            jax.ShapeDtypeStruct((n,), jnp.float32),
            jax.ShapeDtypeStruct((n,), jnp.float32),
            jax.ShapeDtypeStruct((LANES,), jnp.float32),
        ],
    )(x, W_fc, a_l, a_r)

    ch_tot = -(-e_cnt // (NS * CHUNK))
    ch0 = max(1, round(ch_tot * 0.54))
    ch1 = ch_tot - ch0
    chmax = max(ch0, ch1)
    counts = np.array([ch0] * NS + [ch1] * NS) * CHUNK
    offs = np.concatenate([[0], np.cumsum(counts)])[:NW]
    pos = offs[:, None] + np.arange(chmax * CHUNK)[None, :]
    valid = ((np.arange(chmax * CHUNK)[None, :] < counts[:, None])
             & (pos < e_cnt))
    pos_c = jnp.asarray(np.minimum(pos, e_cnt - 1))
    valid = jnp.asarray(valid)
    pk_full = (edge_index[0] << PKBITS) | edge_index[1]
    pk = jnp.where(valid, jnp.take(pk_full, pos_c),
                   0).reshape(NW, chmax, CHUNK)
    wgt = jnp.where(valid, jnp.take(edge_weight, pos_c),
                    -1.0).reshape(NW, chmax, CHUNK)

    n_pad = -(-n // (NS * CHUNK)) * NS * CHUNK
    assert n_pad < (1 << PKBITS)
    hp, dp = _make_sc_kernel(n, d, n_pad, ch0, ch1)(z, sl, sr, pk, wgt, cvec)

    out = pl.pallas_call(
        functools.partial(_post_body, n),
        out_shape=jax.ShapeDtypeStruct((n, d), jnp.float32),
    )(hp, dp)
    return out


# speedup vs baseline: 1.3292x; 1.2374x over previous
import functools

import jax
import jax.numpy as jnp
import numpy as np
from jax import lax
from jax.experimental import pallas as pl
from jax.experimental.pallas import tpu as pltpu
from jax.experimental.pallas import tpu_sc as plsc

NC = 2
NS = 16
NW = NC * NS
LANES = 16
CHUNK = 128
PKBITS = 14


def _pre_body(x_ref, w_ref, al_ref, ar_ref, z_ref, sl_ref, sr_ref, cv_ref):
    x = x_ref[...]
    z = lax.dot_general(x, w_ref[...], (((1,), (1,)), ((), ())),
                        preferred_element_type=jnp.float32)
    z_ref[...] = z
    sl = jnp.sum(z * al_ref[...][None, :], axis=1)
    sr = jnp.sum(z * ar_ref[...][None, :], axis=1)
    sl_ref[...] = sl
    sr_ref[...] = sr
    c_sh = jnp.maximum(jnp.max(sl) + jnp.max(sr), 0.0)
    cv_ref[...] = jnp.full((LANES,), c_sh, jnp.float32)


def _post_body(n, hp_ref, dp_ref, o_ref):
    den = dp_ref[0, :n] + dp_ref[1, :n]
    den = jnp.where(den == 0.0, 1.0, den)
    h = hp_ref[0, :n, :] + hp_ref[1, :n, :]
    o_ref[...] = h / den[:, None]


def _make_sc_kernel(n, d, n_pad, ch0, ch1):
    rows_per_tile = n_pad // NS
    zcopies = rows_per_tile // CHUNK
    chmax = max(ch0, ch1)

    mesh = plsc.VectorSubcoreMesh(core_axis_name="c", subcore_axis_name="s")

    @functools.partial(
        pl.kernel,
        out_type=[
            jax.ShapeDtypeStruct((NC, n_pad, d), jnp.float32),
            jax.ShapeDtypeStruct((NC, n_pad), jnp.float32),
        ],
        mesh=mesh,
        scratch_types=[
            pltpu.VMEM((chmax, CHUNK), jnp.int32),
            pltpu.VMEM((chmax, CHUNK), jnp.float32),
            pltpu.VMEM((CHUNK,), jnp.int32),
            pltpu.VMEM((CHUNK,), jnp.int32),
            pltpu.VMEM((CHUNK,), jnp.float32),
            pltpu.VMEM((CHUNK,), jnp.float32),
            pltpu.VMEM((CHUNK,), jnp.float32),
            pltpu.VMEM((CHUNK, d), jnp.float32),
            pltpu.VMEM((LANES,), jnp.float32),
            pltpu.VMEM_SHARED((n_pad, d), jnp.float32),
            pltpu.VMEM_SHARED((n_pad,), jnp.float32),
            pltpu.SemaphoreType.DMA,
            pltpu.SemaphoreType.DMA,
        ],
    )
    def sc_kernel(z_hbm, sl_hbm, sr_hbm, pk_hbm, w_hbm, cv_hbm,
                  h_out, den_out,
                  pk_v, w_v, src_c, dst_c, slg, srg, ex_c, rows, cv_v,
                  h_sh, den_sh, sem_s, sem_z):
        c = lax.axis_index("c")
        s = lax.axis_index("s")
        w_id = c * NS + s
        base = s * rows_per_tile

        pltpu.sync_copy(cv_hbm, cv_v)
        c_sh = cv_v[...]
        pltpu.sync_copy(pk_hbm.at[w_id], pk_v)
        pltpu.sync_copy(w_hbm.at[w_id], w_v)

        def zrow(r, _):
            for f in range(d // LANES):
                rows[r, pl.ds(f * LANES, LANES)] = jnp.zeros(
                    (LANES,), jnp.float32)
            return 0
        lax.fori_loop(0, CHUNK, zrow, 0)
        for q in range(zcopies):
            pltpu.sync_copy(rows, h_sh.at[pl.ds(base + q * CHUNK, CHUNK)])
            pltpu.sync_copy(rows.at[0],
                            den_sh.at[pl.ds(base + q * CHUNK, CHUNK)])

        plsc.subcore_barrier()

        def cbody(j, _):
            for k in range(CHUNK // LANES):
                v = pk_v[j, pl.ds(k * LANES, LANES)]
                src_c[pl.ds(k * LANES, LANES)] = lax.shift_right_logical(
                    v, PKBITS)
                dst_c[pl.ds(k * LANES, LANES)] = lax.bitwise_and(
                    v, (1 << PKBITS) - 1)
            zcp = pltpu.async_copy(z_hbm.at[src_c], rows, sem_z)
            ga = pltpu.async_copy(sl_hbm.at[src_c], slg, sem_s)
            gb = pltpu.async_copy(sr_hbm.at[dst_c], srg, sem_s)
            ga.wait()
            gb.wait()
            for k in range(CHUNK // LANES):
                wk = w_v[j, pl.ds(k * LANES, LANES)]
                raw = (slg[pl.ds(k * LANES, LANES)]
                       + srg[pl.ds(k * LANES, LANES)])
                e = wk * jnp.maximum(raw, 0.01 * raw)
                ex = jnp.where(wk >= 0.0, jnp.exp(e - c_sh), 0.0)
                ex_c[pl.ds(k * LANES, LANES)] = ex
            pltpu.sync_copy(ex_c, den_sh.at[dst_c], add=True)
            zcp.wait()
            def rblk(k, _):
                exk = ex_c[pl.ds(k * LANES, LANES)]
                for r in range(LANES):
                    a = exk[r]
                    row = k * LANES + r
                    for f in range(d // LANES):
                        v = rows[row, pl.ds(f * LANES, LANES)]
                        rows[row, pl.ds(f * LANES, LANES)] = v * a
                return 0
            lax.fori_loop(0, CHUNK // LANES, rblk, 0)
            pltpu.sync_copy(rows, h_sh.at[dst_c], add=True)
            return 0
        nch = jnp.where(c == 0, ch0, ch1)
        lax.fori_loop(0, nch, cbody, 0)

        plsc.subcore_barrier()

        pltpu.sync_copy(h_sh.at[pl.ds(base, rows_per_tile)],
                        h_out.at[c, pl.ds(base, rows_per_tile)])
        pltpu.sync_copy(den_sh.at[pl.ds(base, rows_per_tile)],
                        den_out.at[c, pl.ds(base, rows_per_tile)])

    return sc_kernel


def kernel(x, edge_index, edge_weight, W_fc, W_attn):
    n, d_in = x.shape
    d = W_fc.shape[0]
    e_cnt = edge_index.shape[1]
    assert n % LANES == 0 and d % LANES == 0

    a_l = W_attn[0, :d]
    a_r = W_attn[0, d:]

    z, sl, sr, cvec = pl.pallas_call(
        _pre_body,
        out_shape=[
            jax.ShapeDtypeStruct((n, d), jnp.float32),
            jax.ShapeDtypeStruct((n,), jnp.float32),
            jax.ShapeDtypeStruct((n,), jnp.float32),
            jax.ShapeDtypeStruct((LANES,), jnp.float32),
        ],
    )(x, W_fc, a_l, a_r)

    ch_tot = -(-e_cnt // (NS * CHUNK))
    ch0 = max(1, round(ch_tot * 0.54))
    ch1 = ch_tot - ch0
    chmax = max(ch0, ch1)
    cut = NS * ch0 * CHUNK

    def part(arr, pad_value):
        a0 = arr[:cut].reshape(NS, ch0, CHUNK)
        a0 = jnp.pad(a0, ((0, 0), (0, chmax - ch0), (0, 0)),
                     constant_values=pad_value)
        tail = NS * ch1 * CHUNK - (e_cnt - cut)
        a1 = jnp.pad(arr[cut:], (0, tail),
                     constant_values=pad_value).reshape(NS, ch1, CHUNK)
        a1 = jnp.pad(a1, ((0, 0), (0, chmax - ch1), (0, 0)),
                     constant_values=pad_value)
        return jnp.concatenate([a0, a1], axis=0)

    pk_full = (edge_index[0] << PKBITS) | edge_index[1]
    pk = part(pk_full, 0)
    wgt = part(edge_weight, -1.0)

    n_pad = -(-n // (NS * CHUNK)) * NS * CHUNK
    assert n_pad < (1 << PKBITS)
    hp, dp = _make_sc_kernel(n, d, n_pad, ch0, ch1)(z, sl, sr, pk, wgt, cvec)

    out = pl.pallas_call(
        functools.partial(_post_body, n),
        out_shape=jax.ShapeDtypeStruct((n, d), jnp.float32),
    )(hp, dp)
    return out
